# baseline (speedup 1.0000x reference)
import jax
import jax.numpy as jnp
from jax.experimental import pallas as pl
from jax.experimental.pallas import tpu as pltpu

_BETA = 0.001
_BH = 200


def _body(x_ref, w_ref, bias_ref, adj_a_ref, adj_b_ref, out_ref, sup_ref):
    i = pl.program_id(0)

    @pl.when(i == 0)
    def _compute_support():
        xb = x_ref[...].astype(jnp.bfloat16)
        wb = w_ref[...].astype(jnp.bfloat16)
        sup_ref[...] = jnp.dot(
            xb, wb, preferred_element_type=jnp.float32
        ).astype(jnp.bfloat16)

    sup = sup_ref[...]
    bias = bias_ref[...]
    for half, adj_ref in enumerate((adj_a_ref, adj_b_ref)):
        a = adj_ref[...].astype(jnp.bfloat16)
        acc = jnp.dot(a, sup, preferred_element_type=jnp.float32)
        x_blk = x_ref[pl.ds((2 * i + half) * _BH, _BH), :]
        out = _BETA * x_blk + (1.0 - _BETA) * acc
        norm = jnp.sqrt(jnp.sum(out * out, axis=1, keepdims=True))
        out = out / jnp.maximum(norm, 1e-12)
        out_ref[pl.ds(half * _BH, _BH), :] = out + bias


def kernel(input, adj, weight, bias):
    n, d = input.shape
    bh = _BH
    out = pl.pallas_call(
        _body,
        grid=(n // (2 * bh),),
        in_specs=[
            pl.BlockSpec((n, d), lambda m: (0, 0)),
            pl.BlockSpec((d, d), lambda m: (0, 0)),
            pl.BlockSpec((1, d), lambda m: (0, 0)),
            pl.BlockSpec((bh, n), lambda m: (2 * m, 0)),
            pl.BlockSpec((bh, n), lambda m: (2 * m + 1, 0)),
        ],
        out_specs=pl.BlockSpec((2 * bh, d), lambda m: (m, 0)),
        out_shape=jax.ShapeDtypeStruct((n, d), jnp.float32),
        scratch_shapes=[pltpu.VMEM((n, d), jnp.bfloat16)],
        compiler_params=pltpu.CompilerParams(
            dimension_semantics=("arbitrary",),
        ),
    )(input, weight, bias.reshape(1, d), adj, adj)
    return out


# speedup vs baseline: 1.0012x; 1.0012x over previous
import jax
import jax.numpy as jnp
from jax.experimental import pallas as pl
from jax.experimental.pallas import tpu as pltpu

_BETA = 0.001
_BM = 200


def _body(x_ref, w_ref, bias_ref, adj_a_ref, adj_b_ref, out_ref, sup_ref):
    i = pl.program_id(0)
    half_n = pl.num_programs(0) * _BM

    @pl.when(i == 0)
    def _compute_support():
        xb = x_ref[...].astype(jnp.bfloat16)
        wb = w_ref[...].astype(jnp.bfloat16)
        sup_ref[...] = jnp.dot(
            xb, wb, preferred_element_type=jnp.float32
        ).astype(jnp.bfloat16)

    sup = sup_ref[...]
    bias = bias_ref[...]
    for half, adj_ref in enumerate((adj_a_ref, adj_b_ref)):
        a = adj_ref[0].astype(jnp.bfloat16)
        acc = jnp.dot(a, sup, preferred_element_type=jnp.float32)
        x_blk = x_ref[pl.ds(half * half_n + i * _BM, _BM), :]
        out = _BETA * x_blk + (1.0 - _BETA) * acc
        norm = jnp.sqrt(jnp.sum(out * out, axis=1, keepdims=True))
        out = out / jnp.maximum(norm, 1e-12)
        out_ref[half, :, :] = out + bias


def kernel(input, adj, weight, bias):
    n, d = input.shape
    bm = _BM
    adj3 = adj.reshape(2, n // 2, n)
    out = pl.pallas_call(
        _body,
        grid=(n // 2 // bm,),
        in_specs=[
            pl.BlockSpec((n, d), lambda m: (0, 0)),
            pl.BlockSpec((d, d), lambda m: (0, 0)),
            pl.BlockSpec((1, d), lambda m: (0, 0)),
            pl.BlockSpec((1, bm, n), lambda m: (0, m, 0)),
            pl.BlockSpec((1, bm, n), lambda m: (1, m, 0)),
        ],
        out_specs=pl.BlockSpec((2, bm, d), lambda m: (0, m, 0)),
        out_shape=jax.ShapeDtypeStruct((2, n // 2, d), jnp.float32),
        scratch_shapes=[pltpu.VMEM((n, d), jnp.bfloat16)],
        compiler_params=pltpu.CompilerParams(
            dimension_semantics=("arbitrary",),
        ),
    )(input, weight, bias.reshape(1, d), adj3, adj3)
    return out.reshape(n, d)
